# baseline (speedup 1.0000x reference)
import functools

import jax
import jax.numpy as jnp
from jax import lax
from jax.experimental import pallas as pl
from jax.experimental.pallas import tpu as pltpu
from jax.experimental.pallas import tpu_sc as plsc


@functools.lru_cache(maxsize=None)
def _make_gather(V, D, batch, seq):
    info = plsc.get_sparse_core_info()
    NC, NS = info.num_cores, info.num_subcores
    NW = NC * NS
    assert batch % NW == 0
    rows_per_w = batch // NW
    C = seq
    SEQ_PAD = (seq + 7) // 8 * 8
    NBUF = 4
    AHEAD = 2
    assert rows_per_w % NBUF == 0 and AHEAD < NBUF
    n_groups = rows_per_w // NBUF
    mesh = plsc.VectorSubcoreMesh(core_axis_name="c", subcore_axis_name="s")

    @functools.partial(
        pl.kernel,
        mesh=mesh,
        compiler_params=pltpu.CompilerParams(use_tc_tiling_on_sc=False),
        out_type=jax.ShapeDtypeStruct((batch, SEQ_PAD, D), jnp.float32),
        scratch_types=[
            pltpu.VMEM((rows_per_w, C), jnp.int32),
            pltpu.VMEM((NBUF, C, D), jnp.float32),
        ]
        + [pltpu.SemaphoreType.DMA] * (2 * NBUF),
    )
    def gather_kernel(table_hbm, idx_hbm, out_hbm, idx_v, rows_v, *sems):
        gsems, wsems = sems[:NBUF], sems[NBUF:]
        wid = lax.axis_index("s") * NC + lax.axis_index("c")
        base = wid * rows_per_w
        pltpu.sync_copy(idx_hbm.at[pl.ds(wid * rows_per_w, rows_per_w)], idx_v)

        def start_gather(g, b):
            return pltpu.async_copy(
                table_hbm.at[idx_v.at[g]], rows_v.at[b], gsems[b]
            )

        def start_writeout(g, b):
            return pltpu.async_copy(
                rows_v.at[b], out_hbm.at[base + g].at[pl.ds(0, C)], wsems[b]
            )

        def wait_writeout(g, b):
            pltpu.make_async_copy(
                rows_v.at[b], out_hbm.at[base + g].at[pl.ds(0, C)], wsems[b]
            ).wait()

        def wait_gather(g, b):
            pltpu.make_async_copy(
                table_hbm.at[idx_v.at[g]], rows_v.at[b], gsems[b]
            ).wait()

        for p in range(AHEAD):
            start_gather(p, p)
        for b in range(NBUF):
            p = b + AHEAD
            if p < rows_per_w:
                if p >= NBUF:
                    wait_writeout(p - NBUF, p % NBUF)
                start_gather(p, p % NBUF)
            wait_gather(b, b)
            start_writeout(b, b)

        def group_body(m):
            g0 = m * NBUF
            for b in range(NBUF):
                g = g0 + b
                p = g + AHEAD
                bp = (b + AHEAD) % NBUF
                wait_writeout(p - NBUF, bp)
                start_gather(p, bp)
                wait_gather(g, b)
                start_writeout(g, b)

        if n_groups > 2:
            pl.loop(1, n_groups - 1)(group_body)

        if n_groups > 1:
            g0 = (n_groups - 1) * NBUF
            for b in range(NBUF):
                g = g0 + b
                p = g + AHEAD
                bp = (b + AHEAD) % NBUF
                if p < rows_per_w:
                    wait_writeout(p - NBUF, bp)
                    start_gather(p, bp)
                wait_gather(g, b)
                start_writeout(g, b)
        for b in range(NBUF):
            g = (n_groups - 1) * NBUF + b
            wait_writeout(g, b)

    return gather_kernel


def kernel(inputs, word_embeddings):
    batch, seq = inputs.shape
    V, D = word_embeddings.shape
    idx = inputs.astype(jnp.int32)
    out = _make_gather(V, D, batch, seq)(word_embeddings, idx)
    return out[:, :seq, :]


# speedup vs baseline: 1.1803x; 1.1803x over previous
import functools

import jax
import jax.numpy as jnp
from jax import lax
from jax.experimental import pallas as pl
from jax.experimental.pallas import tpu as pltpu
from jax.experimental.pallas import tpu_sc as plsc


@functools.lru_cache(maxsize=None)
def _make_gather(V, DP, batch, seq):
    info = plsc.get_sparse_core_info()
    NC, NS = info.num_cores, info.num_subcores
    NW = NC * NS
    assert batch % NW == 0
    rows_per_w = batch // NW
    C = seq
    SEQ_PAD = (seq + 7) // 8 * 8
    NBUF = 4
    AHEAD = 2
    assert rows_per_w % NBUF == 0 and AHEAD < NBUF
    n_groups = rows_per_w // NBUF
    mesh = plsc.VectorSubcoreMesh(core_axis_name="c", subcore_axis_name="s")

    @functools.partial(
        pl.kernel,
        mesh=mesh,
        compiler_params=pltpu.CompilerParams(use_tc_tiling_on_sc=False),
        out_type=jax.ShapeDtypeStruct((batch, SEQ_PAD, DP), jnp.float32),
        scratch_types=[
            pltpu.VMEM((rows_per_w, C), jnp.int32),
            pltpu.VMEM((NBUF, C, DP), jnp.float32),
        ]
        + [pltpu.SemaphoreType.DMA] * (2 * NBUF),
    )
    def gather_kernel(table_hbm, idx_hbm, out_hbm, idx_v, rows_v, *sems):
        gsems, wsems = sems[:NBUF], sems[NBUF:]
        wid = lax.axis_index("s") * NC + lax.axis_index("c")
        base = wid * rows_per_w
        pltpu.sync_copy(idx_hbm.at[pl.ds(wid * rows_per_w, rows_per_w)], idx_v)

        def start_gather(g, b):
            return pltpu.async_copy(
                table_hbm.at[idx_v.at[g]], rows_v.at[b], gsems[b]
            )

        def wait_gather(g, b):
            pltpu.make_async_copy(
                table_hbm.at[idx_v.at[g]], rows_v.at[b], gsems[b]
            ).wait()

        def start_writeout(g, b):
            return pltpu.async_copy(
                rows_v.at[b], out_hbm.at[base + g].at[pl.ds(0, C)], wsems[b]
            )

        def wait_writeout(g, b):
            pltpu.make_async_copy(
                rows_v.at[b], out_hbm.at[base + g].at[pl.ds(0, C)], wsems[b]
            ).wait()

        for p in range(AHEAD):
            start_gather(p, p)
        for b in range(NBUF):
            p = b + AHEAD
            if p < rows_per_w:
                if p >= NBUF:
                    wait_writeout(p - NBUF, p % NBUF)
                start_gather(p, p % NBUF)
            wait_gather(b, b)
            start_writeout(b, b)

        def group_body(m):
            g0 = m * NBUF
            for b in range(NBUF):
                g = g0 + b
                p = g + AHEAD
                bp = (b + AHEAD) % NBUF
                wait_writeout(p - NBUF, bp)
                start_gather(p, bp)
                wait_gather(g, b)
                start_writeout(g, b)

        if n_groups > 2:
            pl.loop(1, n_groups - 1)(group_body)

        if n_groups > 1:
            g0 = (n_groups - 1) * NBUF
            for b in range(NBUF):
                g = g0 + b
                p = g + AHEAD
                bp = (b + AHEAD) % NBUF
                if p < rows_per_w:
                    wait_writeout(p - NBUF, bp)
                    start_gather(p, bp)
                wait_gather(g, b)
                start_writeout(g, b)
        for b in range(NBUF):
            g = (n_groups - 1) * NBUF + b
            wait_writeout(g, b)

    return gather_kernel


def kernel(inputs, word_embeddings):
    batch, seq = inputs.shape
    V, D = word_embeddings.shape
    DP = 128
    idx = inputs.astype(jnp.int32)
    wt = jnp.pad(word_embeddings, ((0, 0), (0, DP - D)))
    out = _make_gather(V, DP, batch, seq)(wt, idx)
    return out[:, :seq, :D]
